# baseline (speedup 1.0000x reference)
import functools

import jax
import jax.numpy as jnp
from jax import lax
from jax.experimental import pallas as pl
from jax.experimental.pallas import tpu as pltpu
from jax.experimental.pallas import tpu_sc as plsc

HID = 128
INTER = 64
DEPTH = 2
NB = 64
NNODE = 50000
NRANGE = 6
RNS = 8704
SROWS = RNS + 8
NPADOUT = NRANGE * RNS
TBW = 128
SW = 128
CH = 64
GR = 128
GROWS = 17
PADV = 1 << 20
BN = 2000
NEG_INF = float('-inf')


def _init_body(h1, h2, h3, w0, b0, w1, b1, w2, b2, o1, o2, o3):
    o1[...] = jnp.dot(h1[...], w0[...], preferred_element_type=jnp.float32) + b0[...]
    o2[...] = jnp.dot(h2[...], w1[...], preferred_element_type=jnp.float32) + b1[...]
    o3[...] = jnp.dot(h3[...], w2[...], preferred_element_type=jnp.float32) + b2[...]


def _init_linears(h1, h2, h3, p):
    n = h1.shape[0]
    grid = n // BN
    row = pl.BlockSpec((BN, HID), lambda i: (i, 0))
    full = lambda a: pl.BlockSpec(a.shape, lambda i: (0,) * a.ndim)
    ws = [p['init_W0'], p['init_b0'].reshape(1, -1), p['init_W1'], p['init_b1'].reshape(1, -1),
          p['init_W2'], p['init_b2'].reshape(1, -1)]
    return pl.pallas_call(
        _init_body,
        grid=(grid,),
        in_specs=[row, row, row] + [full(w) for w in ws],
        out_specs=[row, row, row],
        out_shape=[jax.ShapeDtypeStruct((n, HID), jnp.float32)] * 3,
    )(h1, h2, h3, *ws)


def _tables_body(hc, xc, hn, xn, wa, ba, wb, ta, tb):
    z = jnp.zeros((hc.shape[0], TBW - 80), jnp.float32)
    a = jnp.dot(hc[...], wa[...], preferred_element_type=jnp.float32) + ba[...]
    ta[...] = jnp.concatenate([a, xc[...], z], axis=1)
    b = jnp.dot(hn[...], wb[...], preferred_element_type=jnp.float32)
    tb[...] = jnp.concatenate([b, xn[...], z], axis=1)


def _tables(hc, xc, hn, xn, wa, ba, wb):
    n = hc.shape[0]
    grid = n // BN
    row = pl.BlockSpec((BN, HID), lambda i: (i, 0))
    rx = pl.BlockSpec((BN, 16), lambda i: (i, 0))
    ro = pl.BlockSpec((BN, TBW), lambda i: (i, 0))
    full = lambda a: pl.BlockSpec(a.shape, lambda i: (0,) * a.ndim)
    ba2 = ba.reshape(1, -1)
    return pl.pallas_call(
        _tables_body,
        grid=(grid,),
        in_specs=[row, rx, row, rx, full(wa), full(ba2), full(wb)],
        out_specs=[ro, ro],
        out_shape=[jax.ShapeDtypeStruct((n, TBW), jnp.float32)] * 2,
    )(hc, xc, hn, xn, wa, ba2, wb)


def _node_body(has_up, has_lo, *refs):
    refs = list(refs)
    h = refs.pop(0)
    x = refs.pop(0)
    su = refs.pop(0) if has_up else None
    sl = refs.pop(0) if has_lo else None
    wh = refs.pop(0)
    wu = refs.pop(0) if has_up else None
    bu = refs.pop(0) if has_up else None
    wl = refs.pop(0) if has_lo else None
    bl = refs.pop(0) if has_lo else None
    nb = refs.pop(0)
    m3 = refs.pop(0)
    oh, ox = refs
    acc = jnp.dot(h[...], wh[...], preferred_element_type=jnp.float32) + nb[...]
    xacc = x[...]
    for s, w, b in ((su, wu, bu), (sl, wl, bl)):
        if s is None:
            continue
        sv = s[...]
        cnt = sv[:, 67:68]
        cc = jnp.maximum(cnt, 1.0)
        mean = sv[:, 0:64] / cc
        term = jnp.dot(mean, w[...], preferred_element_type=jnp.float32) + b[...]
        acc = acc + jnp.where(cnt > 0.0, term, 0.0)
        xacc = xacc + (sv[:, 64:80] / cc) * m3[...]
    oh[...] = acc
    ox[...] = xacc


def _node_update(h, x, su, sl, wh, wu, bu, wl, bl, nb, m3):
    n = h.shape[0]
    grid = n // BN
    row = pl.BlockSpec((BN, HID), lambda i: (i, 0))
    rx = pl.BlockSpec((BN, 16), lambda i: (i, 0))
    ro = pl.BlockSpec((BN, SW), lambda i: (i, 0))
    full = lambda a: pl.BlockSpec(a.shape, lambda i: (0,) * a.ndim)
    ins = [h, x]
    specs = [row, rx]
    if su is not None:
        ins.append(su)
        specs.append(ro)
    if sl is not None:
        ins.append(sl)
        specs.append(ro)
    small = [wh]
    if su is not None:
        small += [wu, bu.reshape(1, -1)]
    if sl is not None:
        small += [wl, bl.reshape(1, -1)]
    small += [nb.reshape(1, -1), m3]
    ins += small
    specs += [full(a) for a in small]
    return pl.pallas_call(
        functools.partial(_node_body, su is not None, sl is not None),
        grid=(grid,),
        in_specs=specs,
        out_specs=[row, rx],
        out_shape=[jax.ShapeDtypeStruct((n, HID), jnp.float32),
                   jax.ShapeDtypeStruct((n, 16), jnp.float32)],
    )(*ins)


def _pool_body(h1, i1, h2, i2, h3, i3, w1, b1, w2, b2, w3, b3, out,
               m1s, m2s, s3s, c3s):
    pid = pl.program_id(0)
    nsteps = pl.num_programs(0)

    @pl.when(pid == 0)
    def _():
        m1s[...] = jnp.full(m1s.shape, NEG_INF, jnp.float32)
        m2s[...] = jnp.full(m2s.shape, NEG_INF, jnp.float32)
        s3s[...] = jnp.zeros(s3s.shape, jnp.float32)
        c3s[...] = jnp.zeros(c3s.shape, jnp.float32)

    def seg_max(ids_ref, href, acc):
        ids = ids_ref[...]
        hv = href[...]
        lo = jnp.min(ids)
        hi = jnp.max(ids)

        def body(sv, _):
            mask = ids == sv
            contrib = jnp.max(jnp.where(mask, hv, NEG_INF), axis=0, keepdims=True)
            cur = acc[pl.ds(sv, 1), :]
            acc[pl.ds(sv, 1), :] = jnp.maximum(cur, contrib)
            return 0

        lax.fori_loop(lo, hi + 1, body, 0)

    seg_max(i1, h1, m1s)
    seg_max(i2, h2, m2s)
    ids3 = i3[0]
    oh = (lax.broadcasted_iota(jnp.int32, (NB, BN), 0) == ids3).astype(jnp.float32)
    s3s[...] += jnp.dot(oh, h3[...], preferred_element_type=jnp.float32)
    c3s[...] += jnp.sum(oh, axis=1, keepdims=True) * jnp.ones((1, HID), jnp.float32)

    @pl.when(pid == nsteps - 1)
    def _():
        h3b = s3s[...] / jnp.maximum(c3s[...], 1.0)
        H = jnp.concatenate([m1s[...], m2s[...], h3b], axis=1)
        z = jnp.dot(H, w1[...], preferred_element_type=jnp.float32) + b1[...]
        z = jnp.dot(z, w2[...], preferred_element_type=jnp.float32) + b2[...]
        z = jnp.maximum(z, 0.0)
        z = jnp.dot(z, w3[...], preferred_element_type=jnp.float32) + b3[...]
        zm = jnp.max(z, axis=1, keepdims=True)
        ez = jnp.exp(z - zm)
        lse = jnp.log(jnp.sum(ez, axis=1, keepdims=True)) + zm
        out[...] = z - lse


def _pool_cls(h1, b1, h2, b2, h3, b3, p):
    n = h1.shape[0]
    grid = n // BN
    row = pl.BlockSpec((BN, HID), lambda i: (i, 0))
    ri = pl.BlockSpec((BN, 1), lambda i: (i, 0))
    ri3 = pl.BlockSpec((1, 1, BN), lambda i: (i, 0, 0))
    full = lambda a: pl.BlockSpec(a.shape, lambda i: (0,) * a.ndim)
    i1 = b1.reshape(n, 1)
    i2 = b2.reshape(n, 1)
    i3 = b3.reshape(grid, 1, BN)
    ws = [p['cls_W1'], p['cls_b1'].reshape(1, -1), p['cls_W2'], p['cls_b2'].reshape(1, -1),
          p['cls_W3'], p['cls_b3'].reshape(1, -1)]
    return pl.pallas_call(
        _pool_body,
        grid=(grid,),
        in_specs=[row, ri, row, ri, row, ri3] + [full(w) for w in ws],
        out_specs=pl.BlockSpec((NB, 10), lambda i: (0, 0)),
        out_shape=jax.ShapeDtypeStruct((NB, 10), jnp.float32),
        scratch_shapes=[pltpu.VMEM((NB, HID), jnp.float32)] * 4,
    )(h1, i1, h2, i2, h3, i3, *ws)


def _sc_edge_body(epad, scan_sizes, ta, tb, ic, jc, cv, out,
                  S, ci, cj, cil, ga0, ga1, gb0, gb1, ob, sbi, sbj, cvm,
                  sm0, sm1):
    c = lax.axis_index("c")
    s = lax.axis_index("s")
    ept = epad // 16
    t0 = s * ept
    gab = ((ga0, gb0, sm0), (ga1, gb1, sm1))

    pltpu.sync_copy(cv, cvm)
    w1d = [cvm[pl.ds(f * 16, 16)] for f in range(4)]
    vv = [cvm[pl.ds(64 + f * 16, 16)] for f in range(4)]
    s0 = cvm[pl.ds(128, 16)][0]
    zero16 = jnp.zeros((16,), jnp.float32)
    one16f = jnp.full((16,), 1.0, jnp.float32)
    iota16 = lax.iota(jnp.int32, 16)
    lane3 = jnp.where(iota16 == jnp.full((16,), 3, jnp.int32), one16f, zero16)
    sh7 = jnp.full((16,), 7, jnp.int32)
    m127 = jnp.full((16,), GR - 1, jnp.int32)
    one16 = jnp.full((16,), 1, jnp.int32)
    zero16i = jnp.zeros((16,), jnp.int32)
    rn16 = jnp.full((16,), RNS, jnp.int32)

    def start(ch, b):
        ga, gb, sm = gab[b]
        pltpu.async_copy(ta.at[ci.at[pl.ds(ch * CH, CH)]], ga, sm)
        pltpu.async_copy(tb.at[cj.at[pl.ds(ch * CH, CH)]], gb, sm)

    def finish(ch, nch, b):
        ga, gb, sm = gab[b]
        pltpu.make_async_copy(ta.at[ci.at[pl.ds(ch * CH, CH)]], ga, sm).wait()
        pltpu.make_async_copy(tb.at[cj.at[pl.ds(ch * CH, CH)]], gb, sm).wait()
        obo = (ch & 1) * CH

        def edge(e, _):
            xc = ga[e, pl.ds(64, 16)]
            xn = gb[e, pl.ds(64, 16)]
            rel = xc - xn
            d2 = jnp.full((16,), jnp.sum(rel * rel), jnp.float32)
            acc = None
            for f in range(4):
                m1 = jnp.maximum(ga[e, pl.ds(f * 16, 16)] + gb[e, pl.ds(f * 16, 16)]
                                 + d2 * w1d[f], zero16)
                ob[obo + e, pl.ds(f * 16, 16)] = m1
                t = m1 * vv[f]
                acc = t if acc is None else acc + t
            coeff = jnp.full((16,), jnp.sum(acc) + s0, jnp.float32)
            ob[obo + e, pl.ds(64, 16)] = rel * coeff + lane3
            return 0

        lax.fori_loop(0, CH, edge, 0)

        @pl.when(((ch & 1) == 1) | (ch == nch - 1))
        def _():
            pltpu.sync_copy(ob, S.at[cil.at[ch >> 1]], add=True)

    def rng(q, _):
        base = (3 * c + q) * RNS

        def zrow(i, _):
            for f in range(SW // 16):
                ob[i, pl.ds(f * 16, 16)] = zero16
            return 0
        lax.fori_loop(0, GR, zrow, 0)
        for k in range(5):
            st = k * 16 + s
            @pl.when(st < RNS // GR)
            def _():
                pltpu.sync_copy(ob, S.at[pl.ds(st * GR, GR)])
        @pl.when(s == 1)
        def _():
            pltpu.sync_copy(ob.at[pl.ds(0, 8)], S.at[pl.ds(RNS, 8)])
        plsc.subcore_barrier()

        lo16 = jnp.full((16,), base, jnp.int32)
        hi16 = lo16 + rn16
        off = 0
        for size in scan_sizes:
            pltpu.sync_copy(ic.at[pl.ds(t0 + off, size)], sbi.at[pl.ds(0, size)])
            pltpu.sync_copy(jc.at[pl.ds(t0 + off, size)], sbj.at[pl.ds(0, size)])

            def scan_body(v, cnt):
                iv = sbi[pl.ds(v * 16, 16)]
                jv = sbj[pl.ds(v * 16, 16)]
                mask = (iv >= lo16) & (iv < hi16)
                mi = jnp.where(mask, one16, zero16i)
                pos = jnp.full((16,), cnt, jnp.int32) + lax.cumsum(mi) - one16
                plsc.store_scatter(ci, [pos], iv, mask=mask)
                plsc.store_scatter(cj, [pos], jv, mask=mask)
                prow = lax.shift_right_logical(pos, sh7)
                pcol = pos & m127
                plsc.store_scatter(cil, [prow, pcol], iv - lo16, mask=mask)
                return cnt + jnp.sum(mi)

            cnt = lax.fori_loop(0, size // 16, scan_body, jnp.int32(0))
            off += size

            for k in range(GR // 16):
                pos = jnp.full((16,), cnt + k * 16, jnp.int32) + iota16
                prow = lax.shift_right_logical(pos, sh7)
                pcol = pos & m127
                plsc.store_scatter(ci, [pos], zero16i)
                plsc.store_scatter(cj, [pos], zero16i)
                plsc.store_scatter(cil, [prow, pcol], rn16)
            nch = (cnt + CH - 1) // CH

            @pl.when(nch > 0)
            def _():
                start(jnp.int32(0), 0)

            def pair(g, _):
                for b in range(2):
                    ch = g * 2 + b

                    @pl.when(ch < nch)
                    def _():
                        @pl.when(ch + 1 < nch)
                        def _():
                            start(ch + 1, b ^ 1)
                        finish(ch, nch, b)
                return 0

            lax.fori_loop(0, (nch + 1) // 2, pair, 0)
        plsc.subcore_barrier()

        for k in range(5):
            st = k * 16 + s
            @pl.when(st < RNS // GR)
            def _():
                pltpu.sync_copy(S.at[pl.ds(st * GR, GR)],
                                out.at[pl.ds(base + st * GR, GR)])
        plsc.subcore_barrier()
        return 0

    lax.fori_loop(0, 3, rng, 0)


@functools.lru_cache(maxsize=None)
def _make_sc_edge(epad):
    ept = epad // 16
    nfull, tail = divmod(ept, 2048)
    scan_sizes = [2048] * nfull + ([tail] if tail else [])
    scan_max = max(scan_sizes)
    mesh = plsc.VectorSubcoreMesh(core_axis_name="c", subcore_axis_name="s")
    return pl.kernel(
        functools.partial(_sc_edge_body, epad, tuple(scan_sizes)),
        mesh=mesh,
        compiler_params=pltpu.CompilerParams(needs_layout_passes=False),
        out_type=jax.ShapeDtypeStruct((NPADOUT, SW), jnp.float32),
        scratch_types=[
            pltpu.VMEM_SHARED((SROWS, SW), jnp.float32),
            pltpu.VMEM((GROWS * GR,), jnp.int32),
            pltpu.VMEM((GROWS * GR,), jnp.int32),
            pltpu.VMEM((GROWS, GR), jnp.int32),
            pltpu.VMEM((CH, TBW), jnp.float32),
            pltpu.VMEM((CH, TBW), jnp.float32),
            pltpu.VMEM((CH, TBW), jnp.float32),
            pltpu.VMEM((CH, TBW), jnp.float32),
            pltpu.VMEM((GR, SW), jnp.float32),
            pltpu.VMEM((scan_max,), jnp.int32),
            pltpu.VMEM((scan_max,), jnp.int32),
            pltpu.VMEM((160,), jnp.float32),
            pltpu.SemaphoreType.DMA,
            pltpu.SemaphoreType.DMA,
        ],
    )


def _edge_pass(tatb, icjc, consts):
    ta, tb = tatb
    ic, jc = icjc
    return _make_sc_edge(ic.shape[0])(ta, tb, ic, jc, consts)


def _prep(p):
    prep = {}
    for i in range(DEPTH):
        for j in range(3):
            ndir = (1 if j < 2 else 0) + (1 if j > 0 else 0)
            nw = p['d%dr%d_node_W' % (i, j)]
            col = 128
            for d, pre in (('up', 'd%dr%d_up' % (i, j)), ('lo', 'd%dr%d_lo' % (i, j))):
                if (d == 'up' and j == 2) or (d == 'lo' and j == 0):
                    continue
                w1 = p[pre + '_W1']
                w2 = p[pre + '_W2']
                cw = p[pre + '_cW'][:, 0]
                v = w2 @ cw
                s0 = p[pre + '_b2'] @ cw + p[pre + '_cb'][0]
                consts = jnp.concatenate([w1[256], v, s0[None], jnp.zeros(31, jnp.float32)])
                nslice = nw[col:col + 64]
                col += 64
                prep[pre] = dict(
                    wa=w1[:128], ba=p[pre + '_b1'], wb=w1[128:256],
                    consts=consts,
                    wf=w2 @ nslice,
                    bf=p[pre + '_b2'] @ nslice,
                )
            prep['node%d%d' % (i, j)] = dict(wh=nw[:128], nb=p['d%dr%d_node_b' % (i, j)])
    return prep


def kernel(h_1, h_2, h_3, x_1, x_2, x_3, b_1, b_2, batch1, batch2, batch3, params):
    p = params
    prep = _prep(p)
    m3 = jnp.concatenate([jnp.ones((1, 3), jnp.float32), jnp.zeros((1, 13), jnp.float32)], axis=1)

    def padx(x):
        return jnp.pad(x, ((0, 0), (0, 13)))

    def pade(e):
        pad = (-e.shape[0]) % 256
        return jnp.pad(e, (0, pad), constant_values=PADV)

    b1r0 = pade(b_1[0])
    b1r1 = pade(b_1[1])
    b2r0 = pade(b_2[0])
    b2r1 = pade(b_2[1])

    hs = list(_init_linears(h_1, h_2, h_3, p))
    xs = [padx(x_1), padx(x_2), padx(x_3)]

    for i in range(DEPTH):
        dirs = [
            ('d%dr0_up' % i, 0, 1, b1r0, b1r1),
            ('d%dr1_lo' % i, 1, 0, b1r1, b1r0),
            ('d%dr1_up' % i, 1, 2, b2r0, b2r1),
            ('d%dr2_lo' % i, 2, 1, b2r1, b2r0),
        ]
        S = {}
        for pre, cidx, nidx, ic, jc in dirs:
            w = prep[pre]
            ta, tb = _tables(hs[cidx], xs[cidx], hs[nidx], xs[nidx],
                             w['wa'], w['ba'], w['wb'])
            S[pre] = _edge_pass((ta, tb), (ic, jc), w['consts'])
        new_h, new_x = [], []
        for j in range(3):
            nd = prep['node%d%d' % (i, j)]
            su = sl = wu = bu = wl = bl = None
            if j < 2:
                pre = 'd%dr%d_up' % (i, j)
                su, wu, bu = S[pre], prep[pre]['wf'], prep[pre]['bf']
            if j > 0:
                pre = 'd%dr%d_lo' % (i, j)
                sl, wl, bl = S[pre], prep[pre]['wf'], prep[pre]['bf']
            nh, nx = _node_update(hs[j], xs[j], su, sl, nd['wh'], wu, bu, wl, bl,
                                  nd['nb'], m3)
            new_h.append(nh)
            new_x.append(nx)
        hs, xs = new_h, new_x

    return _pool_cls(hs[0], batch1, hs[1], batch2, hs[2], batch3, p)


# speedup vs baseline: 2.0160x; 2.0160x over previous
import functools

import jax
import jax.numpy as jnp
from jax import lax
from jax.experimental import pallas as pl
from jax.experimental.pallas import tpu as pltpu
from jax.experimental.pallas import tpu_sc as plsc

HID = 128
INTER = 64
DEPTH = 2
NB = 64
NNODE = 50000
NRANGE = 6
RNS = 8704
SROWS = RNS + 8
NPADOUT = NRANGE * RNS
TBW = 128
SW = 128
CH = 64
CROWS = 64
PADV = 1 << 20
BN = 2000
NEG_INF = float('-inf')


def _init_body(h1, h2, h3, w0, b0, w1, b1, w2, b2, o1, o2, o3):
    o1[...] = jnp.dot(h1[...], w0[...], preferred_element_type=jnp.float32) + b0[...]
    o2[...] = jnp.dot(h2[...], w1[...], preferred_element_type=jnp.float32) + b1[...]
    o3[...] = jnp.dot(h3[...], w2[...], preferred_element_type=jnp.float32) + b2[...]


def _init_linears(h1, h2, h3, p):
    n = h1.shape[0]
    grid = n // BN
    row = pl.BlockSpec((BN, HID), lambda i: (i, 0))
    full = lambda a: pl.BlockSpec(a.shape, lambda i: (0,) * a.ndim)
    ws = [p['init_W0'], p['init_b0'].reshape(1, -1), p['init_W1'], p['init_b1'].reshape(1, -1),
          p['init_W2'], p['init_b2'].reshape(1, -1)]
    return pl.pallas_call(
        _init_body,
        grid=(grid,),
        in_specs=[row, row, row] + [full(w) for w in ws],
        out_specs=[row, row, row],
        out_shape=[jax.ShapeDtypeStruct((n, HID), jnp.float32)] * 3,
    )(h1, h2, h3, *ws)


def _tables_body(hc, xc, hn, xn, wa, ba, wb, ta, tb):
    z = jnp.zeros((hc.shape[0], TBW - 80), jnp.float32)
    a = jnp.dot(hc[...], wa[...], preferred_element_type=jnp.float32) + ba[...]
    ta[...] = jnp.concatenate([a, xc[...], z], axis=1)
    b = jnp.dot(hn[...], wb[...], preferred_element_type=jnp.float32)
    tb[...] = jnp.concatenate([b, xn[...], z], axis=1)


def _tables(hc, xc, hn, xn, wa, ba, wb):
    n = hc.shape[0]
    grid = n // BN
    row = pl.BlockSpec((BN, HID), lambda i: (i, 0))
    rx = pl.BlockSpec((BN, 16), lambda i: (i, 0))
    ro = pl.BlockSpec((BN, TBW), lambda i: (i, 0))
    full = lambda a: pl.BlockSpec(a.shape, lambda i: (0,) * a.ndim)
    ba2 = ba.reshape(1, -1)
    return pl.pallas_call(
        _tables_body,
        grid=(grid,),
        in_specs=[row, rx, row, rx, full(wa), full(ba2), full(wb)],
        out_specs=[ro, ro],
        out_shape=[jax.ShapeDtypeStruct((n, TBW), jnp.float32)] * 2,
    )(hc, xc, hn, xn, wa, ba2, wb)


def _node_body(has_up, has_lo, *refs):
    refs = list(refs)
    h = refs.pop(0)
    x = refs.pop(0)
    su = refs.pop(0) if has_up else None
    sl = refs.pop(0) if has_lo else None
    wh = refs.pop(0)
    wu = refs.pop(0) if has_up else None
    bu = refs.pop(0) if has_up else None
    wl = refs.pop(0) if has_lo else None
    bl = refs.pop(0) if has_lo else None
    nb = refs.pop(0)
    m3 = refs.pop(0)
    oh, ox = refs
    acc = jnp.dot(h[...], wh[...], preferred_element_type=jnp.float32) + nb[...]
    xacc = x[...]
    for s, w, b in ((su, wu, bu), (sl, wl, bl)):
        if s is None:
            continue
        sv = s[...]
        cnt = sv[:, 67:68]
        cc = jnp.maximum(cnt, 1.0)
        mean = sv[:, 0:64] / cc
        term = jnp.dot(mean, w[...], preferred_element_type=jnp.float32) + b[...]
        acc = acc + jnp.where(cnt > 0.0, term, 0.0)
        xacc = xacc + (sv[:, 64:80] / cc) * m3[...]
    oh[...] = acc
    ox[...] = xacc


def _node_update(h, x, su, sl, wh, wu, bu, wl, bl, nb, m3):
    n = h.shape[0]
    grid = n // BN
    row = pl.BlockSpec((BN, HID), lambda i: (i, 0))
    rx = pl.BlockSpec((BN, 16), lambda i: (i, 0))
    ro = pl.BlockSpec((BN, SW), lambda i: (i, 0))
    full = lambda a: pl.BlockSpec(a.shape, lambda i: (0,) * a.ndim)
    ins = [h, x]
    specs = [row, rx]
    if su is not None:
        ins.append(su)
        specs.append(ro)
    if sl is not None:
        ins.append(sl)
        specs.append(ro)
    small = [wh]
    if su is not None:
        small += [wu, bu.reshape(1, -1)]
    if sl is not None:
        small += [wl, bl.reshape(1, -1)]
    small += [nb.reshape(1, -1), m3]
    ins += small
    specs += [full(a) for a in small]
    return pl.pallas_call(
        functools.partial(_node_body, su is not None, sl is not None),
        grid=(grid,),
        in_specs=specs,
        out_specs=[row, rx],
        out_shape=[jax.ShapeDtypeStruct((n, HID), jnp.float32),
                   jax.ShapeDtypeStruct((n, 16), jnp.float32)],
    )(*ins)


def _pool_body(h1, i1, h2, i2, h3, i3, w1, b1, w2, b2, w3, b3, out,
               m1s, m2s, s3s, c3s):
    pid = pl.program_id(0)
    nsteps = pl.num_programs(0)

    @pl.when(pid == 0)
    def _():
        m1s[...] = jnp.full(m1s.shape, NEG_INF, jnp.float32)
        m2s[...] = jnp.full(m2s.shape, NEG_INF, jnp.float32)
        s3s[...] = jnp.zeros(s3s.shape, jnp.float32)
        c3s[...] = jnp.zeros(c3s.shape, jnp.float32)

    def seg_max(ids_ref, href, acc):
        ids = ids_ref[...]
        hv = href[...]
        lo = jnp.min(ids)
        hi = jnp.max(ids)

        def body(sv, _):
            mask = ids == sv
            contrib = jnp.max(jnp.where(mask, hv, NEG_INF), axis=0, keepdims=True)
            cur = acc[pl.ds(sv, 1), :]
            acc[pl.ds(sv, 1), :] = jnp.maximum(cur, contrib)
            return 0

        lax.fori_loop(lo, hi + 1, body, 0)

    seg_max(i1, h1, m1s)
    seg_max(i2, h2, m2s)
    ids3 = i3[0]
    oh = (lax.broadcasted_iota(jnp.int32, (NB, BN), 0) == ids3).astype(jnp.float32)
    s3s[...] += jnp.dot(oh, h3[...], preferred_element_type=jnp.float32)
    c3s[...] += jnp.sum(oh, axis=1, keepdims=True) * jnp.ones((1, HID), jnp.float32)

    @pl.when(pid == nsteps - 1)
    def _():
        h3b = s3s[...] / jnp.maximum(c3s[...], 1.0)
        H = jnp.concatenate([m1s[...], m2s[...], h3b], axis=1)
        z = jnp.dot(H, w1[...], preferred_element_type=jnp.float32) + b1[...]
        z = jnp.dot(z, w2[...], preferred_element_type=jnp.float32) + b2[...]
        z = jnp.maximum(z, 0.0)
        z = jnp.dot(z, w3[...], preferred_element_type=jnp.float32) + b3[...]
        zm = jnp.max(z, axis=1, keepdims=True)
        ez = jnp.exp(z - zm)
        lse = jnp.log(jnp.sum(ez, axis=1, keepdims=True)) + zm
        out[...] = z - lse


def _pool_cls(h1, b1, h2, b2, h3, b3, p):
    n = h1.shape[0]
    grid = n // BN
    row = pl.BlockSpec((BN, HID), lambda i: (i, 0))
    ri = pl.BlockSpec((BN, 1), lambda i: (i, 0))
    ri3 = pl.BlockSpec((1, 1, BN), lambda i: (i, 0, 0))
    full = lambda a: pl.BlockSpec(a.shape, lambda i: (0,) * a.ndim)
    i1 = b1.reshape(n, 1)
    i2 = b2.reshape(n, 1)
    i3 = b3.reshape(grid, 1, BN)
    ws = [p['cls_W1'], p['cls_b1'].reshape(1, -1), p['cls_W2'], p['cls_b2'].reshape(1, -1),
          p['cls_W3'], p['cls_b3'].reshape(1, -1)]
    return pl.pallas_call(
        _pool_body,
        grid=(grid,),
        in_specs=[row, ri, row, ri, row, ri3] + [full(w) for w in ws],
        out_specs=pl.BlockSpec((NB, 10), lambda i: (0, 0)),
        out_shape=jax.ShapeDtypeStruct((NB, 10), jnp.float32),
        scratch_shapes=[pltpu.VMEM((NB, HID), jnp.float32)] * 4,
    )(h1, i1, h2, i2, h3, i3, *ws)


def _sc_edge_body(epad, scan_sizes, ta, tb, ic, jc, cv, out,
                  S, ci, cj, cil, ga, gb, ob, sbi, sbj, cvm, sem):
    c = lax.axis_index("c")
    s = lax.axis_index("s")
    ept = epad // 16
    t0 = s * ept

    pltpu.sync_copy(cv, cvm)
    w1d = [cvm[pl.ds(f * 16, 16)] for f in range(4)]
    vv = [cvm[pl.ds(64 + f * 16, 16)] for f in range(4)]
    s0 = cvm[pl.ds(128, 16)][0]
    zero16 = jnp.zeros((16,), jnp.float32)
    one16f = jnp.full((16,), 1.0, jnp.float32)
    iota16 = lax.iota(jnp.int32, 16)
    lane3 = jnp.where(iota16 == jnp.full((16,), 3, jnp.int32), one16f, zero16)
    sh6 = jnp.full((16,), 6, jnp.int32)
    m63 = jnp.full((16,), 63, jnp.int32)
    mrow = jnp.full((16,), CROWS - 1, jnp.int32)
    one16 = jnp.full((16,), 1, jnp.int32)
    zero16i = jnp.zeros((16,), jnp.int32)
    rn16 = jnp.full((16,), RNS, jnp.int32)

    def chunk(ch, _):
        row = ch & (CROWS - 1)
        ca = pltpu.async_copy(ta.at[ci.at[row]], ga, sem)
        cb = pltpu.async_copy(tb.at[cj.at[row]], gb, sem)
        ca.wait()
        cb.wait()

        def edge(e, _):
            xc = ga[e, pl.ds(64, 16)]
            xn = gb[e, pl.ds(64, 16)]
            rel = xc - xn
            d2 = jnp.full((16,), jnp.sum(rel * rel), jnp.float32)
            acc = None
            for f in range(4):
                m1 = jnp.maximum(ga[e, pl.ds(f * 16, 16)] + gb[e, pl.ds(f * 16, 16)]
                                 + d2 * w1d[f], zero16)
                ob[e, pl.ds(f * 16, 16)] = m1
                t = m1 * vv[f]
                acc = t if acc is None else acc + t
            coeff = jnp.full((16,), jnp.sum(acc) + s0, jnp.float32)
            ob[e, pl.ds(64, 16)] = rel * coeff + lane3
            return 0

        lax.fori_loop(0, CH, edge, 0)
        pltpu.sync_copy(ob, S.at[cil.at[row]], add=True)
        return 0

    for q in range(3):
        base = (3 * c + q) * RNS

        def zrow(i, _):
            for f in range(SW // 16):
                ob[i, pl.ds(f * 16, 16)] = zero16
            return 0
        lax.fori_loop(0, CH, zrow, 0)
        for k in range(9):
            st = k * 16 + s
            @pl.when(st < 136)
            def _():
                pltpu.sync_copy(ob, S.at[pl.ds(st * CH, CH)])
        @pl.when(s == 1)
        def _():
            pltpu.sync_copy(ob.at[pl.ds(0, 8)], S.at[pl.ds(RNS, 8)])
        plsc.subcore_barrier()

        lo16 = jnp.full((16,), base, jnp.int32)
        hi16 = lo16 + rn16
        cnt = jnp.int32(0)
        done = jnp.int32(0)
        off = 0
        for size in scan_sizes:
            pltpu.sync_copy(ic.at[pl.ds(t0 + off, size)], sbi.at[pl.ds(0, size)])
            pltpu.sync_copy(jc.at[pl.ds(t0 + off, size)], sbj.at[pl.ds(0, size)])

            def scan_body(v, cnt):
                iv = sbi[pl.ds(v * 16, 16)]
                jv = sbj[pl.ds(v * 16, 16)]
                mask = (iv >= lo16) & (iv < hi16)
                mi = jnp.where(mask, one16, zero16i)
                pos = jnp.full((16,), cnt, jnp.int32) + lax.cumsum(mi) - one16
                prow = lax.shift_right_logical(pos, sh6) & mrow
                pcol = pos & m63
                plsc.store_scatter(ci, [prow, pcol], iv, mask=mask)
                plsc.store_scatter(cj, [prow, pcol], jv, mask=mask)
                plsc.store_scatter(cil, [prow, pcol], iv - lo16, mask=mask)
                return cnt + jnp.sum(mi)

            cnt = lax.fori_loop(0, size // 16, scan_body, cnt)
            off += size
            nfull = cnt // CH
            lax.fori_loop(done, nfull, chunk, 0)
            done = nfull

        for k in range(CH // 16):
            pos = jnp.full((16,), cnt + k * 16, jnp.int32) + iota16
            prow = lax.shift_right_logical(pos, sh6) & mrow
            pcol = pos & m63
            plsc.store_scatter(ci, [prow, pcol], zero16i)
            plsc.store_scatter(cj, [prow, pcol], zero16i)
            plsc.store_scatter(cil, [prow, pcol], rn16)
        nch = (cnt + CH - 1) // CH
        lax.fori_loop(done, nch, chunk, 0)
        plsc.subcore_barrier()

        for k in range(9):
            st = k * 16 + s
            @pl.when(st < 136)
            def _():
                pltpu.sync_copy(S.at[pl.ds(st * CH, CH)],
                                out.at[pl.ds(base + st * CH, CH)])
        plsc.subcore_barrier()


@functools.lru_cache(maxsize=None)
def _make_sc_edge(epad):
    ept = epad // 16
    nfull, tail = divmod(ept, 2048)
    scan_sizes = [2048] * nfull + ([tail] if tail else [])
    scan_max = max(scan_sizes)
    mesh = plsc.VectorSubcoreMesh(core_axis_name="c", subcore_axis_name="s")
    return pl.kernel(
        functools.partial(_sc_edge_body, epad, tuple(scan_sizes)),
        mesh=mesh,
        compiler_params=pltpu.CompilerParams(needs_layout_passes=False),
        out_type=jax.ShapeDtypeStruct((NPADOUT, SW), jnp.float32),
        scratch_types=[
            pltpu.VMEM_SHARED((SROWS, SW), jnp.float32),
            pltpu.VMEM((CROWS, CH), jnp.int32),
            pltpu.VMEM((CROWS, CH), jnp.int32),
            pltpu.VMEM((CROWS, CH), jnp.int32),
            pltpu.VMEM((CH, TBW), jnp.float32),
            pltpu.VMEM((CH, TBW), jnp.float32),
            pltpu.VMEM((CH, SW), jnp.float32),
            pltpu.VMEM((scan_max,), jnp.int32),
            pltpu.VMEM((scan_max,), jnp.int32),
            pltpu.VMEM((160,), jnp.float32),
            pltpu.SemaphoreType.DMA,
        ],
    )


def _edge_pass(tatb, icjc, consts):
    ta, tb = tatb
    ic, jc = icjc
    return _make_sc_edge(ic.shape[0])(ta, tb, ic, jc, consts)


def _prep(p):
    prep = {}
    for i in range(DEPTH):
        for j in range(3):
            ndir = (1 if j < 2 else 0) + (1 if j > 0 else 0)
            nw = p['d%dr%d_node_W' % (i, j)]
            col = 128
            for d, pre in (('up', 'd%dr%d_up' % (i, j)), ('lo', 'd%dr%d_lo' % (i, j))):
                if (d == 'up' and j == 2) or (d == 'lo' and j == 0):
                    continue
                w1 = p[pre + '_W1']
                w2 = p[pre + '_W2']
                cw = p[pre + '_cW'][:, 0]
                v = w2 @ cw
                s0 = p[pre + '_b2'] @ cw + p[pre + '_cb'][0]
                consts = jnp.concatenate([w1[256], v, s0[None], jnp.zeros(31, jnp.float32)])
                nslice = nw[col:col + 64]
                col += 64
                prep[pre] = dict(
                    wa=w1[:128], ba=p[pre + '_b1'], wb=w1[128:256],
                    consts=consts,
                    wf=w2 @ nslice,
                    bf=p[pre + '_b2'] @ nslice,
                )
            prep['node%d%d' % (i, j)] = dict(wh=nw[:128], nb=p['d%dr%d_node_b' % (i, j)])
    return prep


def kernel(h_1, h_2, h_3, x_1, x_2, x_3, b_1, b_2, batch1, batch2, batch3, params):
    p = params
    prep = _prep(p)
    m3 = jnp.concatenate([jnp.ones((1, 3), jnp.float32), jnp.zeros((1, 13), jnp.float32)], axis=1)

    def padx(x):
        return jnp.pad(x, ((0, 0), (0, 13)))

    def pade(e):
        pad = (-e.shape[0]) % 256
        return jnp.pad(e, (0, pad), constant_values=PADV)

    b1r0 = pade(b_1[0])
    b1r1 = pade(b_1[1])
    b2r0 = pade(b_2[0])
    b2r1 = pade(b_2[1])

    hs = list(_init_linears(h_1, h_2, h_3, p))
    xs = [padx(x_1), padx(x_2), padx(x_3)]

    for i in range(DEPTH):
        dirs = [
            ('d%dr0_up' % i, 0, 1, b1r0, b1r1),
            ('d%dr1_lo' % i, 1, 0, b1r1, b1r0),
            ('d%dr1_up' % i, 1, 2, b2r0, b2r1),
            ('d%dr2_lo' % i, 2, 1, b2r1, b2r0),
        ]
        S = {}
        for pre, cidx, nidx, ic, jc in dirs:
            w = prep[pre]
            ta, tb = _tables(hs[cidx], xs[cidx], hs[nidx], xs[nidx],
                             w['wa'], w['ba'], w['wb'])
            S[pre] = _edge_pass((ta, tb), (ic, jc), w['consts'])
        new_h, new_x = [], []
        for j in range(3):
            nd = prep['node%d%d' % (i, j)]
            su = sl = wu = bu = wl = bl = None
            if j < 2:
                pre = 'd%dr%d_up' % (i, j)
                su, wu, bu = S[pre], prep[pre]['wf'], prep[pre]['bf']
            if j > 0:
                pre = 'd%dr%d_lo' % (i, j)
                sl, wl, bl = S[pre], prep[pre]['wf'], prep[pre]['bf']
            nh, nx = _node_update(hs[j], xs[j], su, sl, nd['wh'], wu, bu, wl, bl,
                                  nd['nb'], m3)
            new_h.append(nh)
            new_x.append(nx)
        hs, xs = new_h, new_x

    return _pool_cls(hs[0], batch1, hs[1], batch2, hs[2], batch3, p)
